# Initial kernel scaffold; baseline (speedup 1.0000x reference)
import jax
import jax.numpy as jnp
from jax.experimental import pallas as pl


def kernel(inp, W):
    raise NotImplementedError("write your pallas kernel here")


# speedup vs baseline: 1.2174x; 1.2174x over previous
import functools

import jax
import jax.numpy as jnp
from jax.experimental import pallas as pl
from jax.experimental.pallas import tpu as pltpu

_TOP_K = 8
_TEMP_INV = 1.0 / 0.3


def _router_body(x_ref, w_ref, idx_ref, score_ref, loss_ref, me_ref, ce_ref,
                 *, n_tokens):
    i = pl.program_id(0)
    n_steps = pl.num_programs(0)
    x = x_ref[...]
    w = w_ref[...]
    logits = jax.lax.dot_general(
        x, w, (((1,), (1,)), ((), ())), preferred_element_type=jnp.float32
    )
    n_expert = logits.shape[1]
    col = jax.lax.broadcasted_iota(jnp.int32, logits.shape, 1)

    lx = logits
    vals = []
    idxs = []
    for _ in range(_TOP_K):
        m = jnp.max(lx, axis=1, keepdims=True)
        am = jnp.min(jnp.where(lx == m, col, n_expert), axis=1, keepdims=True)
        vals.append(m)
        idxs.append(am)
        lx = jnp.where(col == am, -jnp.inf, lx)
    topv = jnp.concatenate(vals, axis=1)
    topi = jnp.concatenate(idxs, axis=1)

    e = jnp.exp(topv - vals[0])
    score_ref[...] = e / jnp.sum(e, axis=1, keepdims=True)
    idx_ref[...] = topi

    t = logits * _TEMP_INV
    t = jnp.exp(t - jnp.max(t, axis=1, keepdims=True))
    p = t / jnp.sum(t, axis=1, keepdims=True)
    me_part = jnp.sum(p, axis=0, keepdims=True)
    ce_part = jnp.sum((col == idxs[0]).astype(jnp.float32), axis=0,
                      keepdims=True)

    @pl.when(i == 0)
    def _init():
        me_ref[...] = jnp.zeros_like(me_ref)
        ce_ref[...] = jnp.zeros_like(ce_ref)

    me_ref[...] += me_part
    ce_ref[...] += ce_part

    @pl.when(i == n_steps - 1)
    def _finalize():
        hot_value = n_expert / n_tokens
        loss = jnp.sum(me_ref[...] * ce_ref[...], axis=1, keepdims=True) * (
            hot_value / n_tokens)
        loss_ref[...] = loss


def kernel(inp, W):
    n_tokens, d_model = inp.shape
    n_expert = W.shape[0]
    tile = 512
    while n_tokens % tile:
        tile //= 2
    grid = n_tokens // tile

    idx, score, loss = pl.pallas_call(
        functools.partial(_router_body, n_tokens=n_tokens),
        grid=(grid,),
        in_specs=[
            pl.BlockSpec((tile, d_model), lambda i: (i, 0)),
            pl.BlockSpec((n_expert, d_model), lambda i: (0, 0)),
        ],
        out_specs=[
            pl.BlockSpec((tile, _TOP_K), lambda i: (i, 0)),
            pl.BlockSpec((tile, _TOP_K), lambda i: (i, 0)),
            pl.BlockSpec((1, 1), lambda i: (0, 0)),
        ],
        out_shape=[
            jax.ShapeDtypeStruct((n_tokens, _TOP_K), jnp.int32),
            jax.ShapeDtypeStruct((n_tokens, _TOP_K), jnp.float32),
            jax.ShapeDtypeStruct((1, 1), jnp.float32),
        ],
        scratch_shapes=[
            pltpu.VMEM((1, n_expert), jnp.float32),
            pltpu.VMEM((1, n_expert), jnp.float32),
        ],
    )(inp, W)
    return idx, score, loss.reshape(())


# speedup vs baseline: 1.4028x; 1.1523x over previous
import functools

import jax
import jax.numpy as jnp
from jax.experimental import pallas as pl
from jax.experimental.pallas import tpu as pltpu

_TOP_K = 8
_TEMP_INV = 1.0 / 0.3


def _router_body(x_ref, w_ref, idx_ref, score_ref, loss_ref, me_ref, ce_ref,
                 *, n_tokens):
    i = pl.program_id(0)
    n_steps = pl.num_programs(0)
    x = x_ref[...]
    w = w_ref[...]
    logits = jax.lax.dot_general(
        x, w, (((1,), (1,)), ((), ())), preferred_element_type=jnp.float32
    )
    n_expert = logits.shape[1]
    col = jax.lax.broadcasted_iota(jnp.int32, logits.shape, 1)

    lx = logits
    vals = []
    idxs = []
    for _ in range(_TOP_K):
        m = jnp.max(lx, axis=1, keepdims=True)
        am = jnp.min(jnp.where(lx == m, col, n_expert), axis=1, keepdims=True)
        vals.append(m)
        idxs.append(am)
        lx = jnp.where(col == am, -jnp.inf, lx)
    topv = jnp.concatenate(vals, axis=1)
    topi = jnp.concatenate(idxs, axis=1)

    e = jnp.exp(topv - vals[0])
    score_ref[...] = e / jnp.sum(e, axis=1, keepdims=True)
    idx_ref[...] = topi

    t = logits * _TEMP_INV
    t = jnp.exp(t - jnp.max(t, axis=1, keepdims=True))
    p = t / jnp.sum(t, axis=1, keepdims=True)
    me_part = jnp.sum(p, axis=0, keepdims=True)
    ce_part = jnp.sum((col == idxs[0]).astype(jnp.float32), axis=0,
                      keepdims=True)

    @pl.when(i == 0)
    def _init():
        me_ref[...] = jnp.zeros_like(me_ref)
        ce_ref[...] = jnp.zeros_like(ce_ref)

    me_ref[...] += me_part
    ce_ref[...] += ce_part

    @pl.when(i == n_steps - 1)
    def _finalize():
        hot_value = n_expert / n_tokens
        loss = jnp.sum(me_ref[...] * ce_ref[...], axis=1, keepdims=True) * (
            hot_value / n_tokens)
        loss_ref[...] = loss


def kernel(inp, W):
    n_tokens, d_model = inp.shape
    n_expert = W.shape[0]
    tile = 1024
    while n_tokens % tile:
        tile //= 2
    grid = n_tokens // tile

    idx, score, loss = pl.pallas_call(
        functools.partial(_router_body, n_tokens=n_tokens),
        grid=(grid,),
        in_specs=[
            pl.BlockSpec((tile, d_model), lambda i: (i, 0)),
            pl.BlockSpec((n_expert, d_model), lambda i: (0, 0)),
        ],
        out_specs=[
            pl.BlockSpec((tile, _TOP_K), lambda i: (i, 0)),
            pl.BlockSpec((tile, _TOP_K), lambda i: (i, 0)),
            pl.BlockSpec((1, 1), lambda i: (0, 0)),
        ],
        out_shape=[
            jax.ShapeDtypeStruct((n_tokens, _TOP_K), jnp.int32),
            jax.ShapeDtypeStruct((n_tokens, _TOP_K), jnp.float32),
            jax.ShapeDtypeStruct((1, 1), jnp.float32),
        ],
        scratch_shapes=[
            pltpu.VMEM((1, n_expert), jnp.float32),
            pltpu.VMEM((1, n_expert), jnp.float32),
        ],
    )(inp, W)
    return idx, score, loss.reshape(())


# speedup vs baseline: 1.4319x; 1.0208x over previous
import functools

import jax
import jax.numpy as jnp
from jax.experimental import pallas as pl
from jax.experimental.pallas import tpu as pltpu

_TOP_K = 8
_TEMP_INV = 1.0 / 0.3


def _router_body(x_ref, w_ref, idx_ref, score_ref, loss_ref, me_ref, ce_ref,
                 *, n_tokens):
    i = pl.program_id(0)
    n_steps = pl.num_programs(0)
    x = x_ref[...]
    w = w_ref[...]
    logits = jax.lax.dot_general(
        x, w, (((1,), (1,)), ((), ())), preferred_element_type=jnp.float32
    )
    n_expert = logits.shape[1]
    col = jax.lax.broadcasted_iota(jnp.int32, logits.shape, 1)

    bits = jax.lax.bitcast_convert_type(logits, jnp.int32)
    okey = jnp.where(bits < 0, bits ^ jnp.int32(0x7FFFFFFF), bits)
    pkey = (okey & jnp.int32(~63)) | (jnp.int32(n_expert - 1) - col)
    pbits = jnp.where(pkey < 0, pkey ^ jnp.int32(0x7FFFFFFF), pkey)
    pf = jax.lax.bitcast_convert_type(pbits, jnp.float32)

    packed = []
    for _ in range(_TOP_K):
        m = jnp.max(pf, axis=1, keepdims=True)
        packed.append(m)
        pf = jnp.where(pf == m, -jnp.inf, pf)
    topv = jnp.concatenate(packed, axis=1)
    tb = jax.lax.bitcast_convert_type(topv, jnp.int32)
    tkey = jnp.where(tb < 0, tb ^ jnp.int32(0x7FFFFFFF), tb)
    topi = jnp.int32(n_expert - 1) - (tkey & jnp.int32(63))
    top1 = topi[:, :1]

    e = jnp.exp(topv - topv[:, :1])
    score_ref[...] = e / jnp.sum(e, axis=1, keepdims=True)
    idx_ref[...] = topi

    t = logits * _TEMP_INV
    t = jnp.exp(t - jnp.max(t, axis=1, keepdims=True))
    p = t / jnp.sum(t, axis=1, keepdims=True)
    me_part = jnp.sum(p, axis=0, keepdims=True)
    ce_part = jnp.sum((col == top1).astype(jnp.float32), axis=0,
                      keepdims=True)

    @pl.when(i == 0)
    def _init():
        me_ref[...] = jnp.zeros_like(me_ref)
        ce_ref[...] = jnp.zeros_like(ce_ref)

    me_ref[...] += me_part
    ce_ref[...] += ce_part

    @pl.when(i == n_steps - 1)
    def _finalize():
        hot_value = n_expert / n_tokens
        loss = jnp.sum(me_ref[...] * ce_ref[...], axis=1, keepdims=True) * (
            hot_value / n_tokens)
        loss_ref[...] = loss


def kernel(inp, W):
    n_tokens, d_model = inp.shape
    n_expert = W.shape[0]
    tile = 1024
    while n_tokens % tile:
        tile //= 2
    grid = n_tokens // tile

    idx, score, loss = pl.pallas_call(
        functools.partial(_router_body, n_tokens=n_tokens),
        grid=(grid,),
        in_specs=[
            pl.BlockSpec((tile, d_model), lambda i: (i, 0)),
            pl.BlockSpec((n_expert, d_model), lambda i: (0, 0)),
        ],
        out_specs=[
            pl.BlockSpec((tile, _TOP_K), lambda i: (i, 0)),
            pl.BlockSpec((tile, _TOP_K), lambda i: (i, 0)),
            pl.BlockSpec((1, 1), lambda i: (0, 0)),
        ],
        out_shape=[
            jax.ShapeDtypeStruct((n_tokens, _TOP_K), jnp.int32),
            jax.ShapeDtypeStruct((n_tokens, _TOP_K), jnp.float32),
            jax.ShapeDtypeStruct((1, 1), jnp.float32),
        ],
        scratch_shapes=[
            pltpu.VMEM((1, n_expert), jnp.float32),
            pltpu.VMEM((1, n_expert), jnp.float32),
        ],
    )(inp, W)
    return idx, score, loss.reshape(())


# speedup vs baseline: 1.4440x; 1.0084x over previous
import functools

import jax
import jax.numpy as jnp
from jax.experimental import pallas as pl
from jax.experimental.pallas import tpu as pltpu

_TOP_K = 8
_TEMP_INV = 1.0 / 0.3


def _router_body(x_ref, w_ref, idx_ref, score_ref, me_ref, ce_ref):
    x = x_ref[...]
    w = w_ref[...]
    logits = jax.lax.dot_general(
        x, w, (((1,), (1,)), ((), ())), preferred_element_type=jnp.float32
    )
    n_expert = logits.shape[1]
    col = jax.lax.broadcasted_iota(jnp.int32, logits.shape, 1)

    bits = jax.lax.bitcast_convert_type(logits, jnp.int32)
    okey = jnp.where(bits < 0, bits ^ jnp.int32(0x7FFFFFFF), bits)
    pkey = (okey & jnp.int32(~63)) | (jnp.int32(n_expert - 1) - col)
    pbits = jnp.where(pkey < 0, pkey ^ jnp.int32(0x7FFFFFFF), pkey)
    pf = jax.lax.bitcast_convert_type(pbits, jnp.float32)

    packed = []
    for _ in range(_TOP_K):
        m = jnp.max(pf, axis=1, keepdims=True)
        packed.append(m)
        pf = jnp.where(pf == m, -jnp.inf, pf)
    topv = jnp.concatenate(packed, axis=1)
    tb = jax.lax.bitcast_convert_type(topv, jnp.int32)
    tkey = jnp.where(tb < 0, tb ^ jnp.int32(0x7FFFFFFF), tb)
    topi = jnp.int32(n_expert - 1) - (tkey & jnp.int32(63))
    top1 = topi[:, :1]

    e = jnp.exp(topv - topv[:, :1])
    score_ref[...] = e / jnp.sum(e, axis=1, keepdims=True)
    idx_ref[...] = topi

    t = logits * _TEMP_INV
    t = jnp.exp(t - jnp.max(t, axis=1, keepdims=True))
    p = t / jnp.sum(t, axis=1, keepdims=True)
    me_ref[0, ...] = jnp.sum(p, axis=0, keepdims=True)
    ce_ref[0, ...] = jnp.sum((col == top1).astype(jnp.float32), axis=0,
                             keepdims=True)


def _loss_body(me_ref, ce_ref, loss_ref, *, n_tokens, n_expert):
    me = jnp.sum(me_ref[:, 0, :], axis=0, keepdims=True)
    ce = jnp.sum(ce_ref[:, 0, :], axis=0, keepdims=True)
    hot_value = n_expert / n_tokens
    loss_ref[...] = jnp.sum(me * ce, axis=1, keepdims=True) * (
        hot_value / n_tokens)


def kernel(inp, W):
    n_tokens, d_model = inp.shape
    n_expert = W.shape[0]
    tile = 1024
    while n_tokens % tile:
        tile //= 2
    grid = n_tokens // tile

    idx, score, me_parts, ce_parts = pl.pallas_call(
        _router_body,
        grid=(grid,),
        in_specs=[
            pl.BlockSpec((tile, d_model), lambda i: (i, 0)),
            pl.BlockSpec((n_expert, d_model), lambda i: (0, 0)),
        ],
        out_specs=[
            pl.BlockSpec((tile, _TOP_K), lambda i: (i, 0)),
            pl.BlockSpec((tile, _TOP_K), lambda i: (i, 0)),
            pl.BlockSpec((1, 1, n_expert), lambda i: (i, 0, 0)),
            pl.BlockSpec((1, 1, n_expert), lambda i: (i, 0, 0)),
        ],
        out_shape=[
            jax.ShapeDtypeStruct((n_tokens, _TOP_K), jnp.int32),
            jax.ShapeDtypeStruct((n_tokens, _TOP_K), jnp.float32),
            jax.ShapeDtypeStruct((grid, 1, n_expert), jnp.float32),
            jax.ShapeDtypeStruct((grid, 1, n_expert), jnp.float32),
        ],
        compiler_params=pltpu.CompilerParams(
            dimension_semantics=("parallel",),
        ),
    )(inp, W)

    loss = pl.pallas_call(
        functools.partial(_loss_body, n_tokens=n_tokens, n_expert=n_expert),
        out_shape=jax.ShapeDtypeStruct((1, 1), jnp.float32),
    )(me_parts, ce_parts)
    return idx, score, loss.reshape(())


# speedup vs baseline: 1.6239x; 1.1246x over previous
import functools

import jax
import jax.numpy as jnp
from jax.experimental import pallas as pl
from jax.experimental.pallas import tpu as pltpu

_TOP_K = 8
_TEMP_INV = 1.0 / 0.3


def _router_body(x_ref, w_ref, idx_ref, score_ref, me_ref, ce_ref):
    x = x_ref[...]
    w = w_ref[...]
    logits = jax.lax.dot_general(
        x, w, (((1,), (1,)), ((), ())), preferred_element_type=jnp.float32
    )
    n_expert = logits.shape[1]
    lt = logits.T
    row = jax.lax.broadcasted_iota(jnp.int32, lt.shape, 0)

    bits = jax.lax.bitcast_convert_type(lt, jnp.int32)
    okey = jnp.where(bits < 0, bits ^ jnp.int32(0x7FFFFFFF), bits)
    pkey = (okey & jnp.int32(~63)) | (jnp.int32(n_expert - 1) - row)
    pbits = jnp.where(pkey < 0, pkey ^ jnp.int32(0x7FFFFFFF), pkey)
    pf = jax.lax.bitcast_convert_type(pbits, jnp.float32)

    packed = []
    ce_mask = None
    for k in range(_TOP_K):
        m = jnp.max(pf, axis=0, keepdims=True)
        packed.append(m)
        hit = pf == m
        if k == 0:
            ce_mask = hit
        pf = jnp.where(hit, -jnp.inf, pf)
    topv = jnp.concatenate(packed, axis=0)
    tb = jax.lax.bitcast_convert_type(topv, jnp.int32)
    tkey = jnp.where(tb < 0, tb ^ jnp.int32(0x7FFFFFFF), tb)
    topi = jnp.int32(n_expert - 1) - (tkey & jnp.int32(63))

    e = jnp.exp(topv - topv[:1, :])
    score_ref[...] = (e / jnp.sum(e, axis=0, keepdims=True)).T
    idx_ref[...] = topi.T

    t = jnp.exp((lt - packed[0]) * _TEMP_INV)
    p = t / jnp.sum(t, axis=0, keepdims=True)
    me_ref[0, ...] = jnp.sum(p, axis=1, keepdims=True).T
    ce_ref[0, ...] = jnp.sum(ce_mask.astype(jnp.float32), axis=1,
                             keepdims=True).T


def _loss_body(me_ref, ce_ref, loss_ref, *, n_tokens, n_expert):
    me = jnp.sum(me_ref[:, 0, :], axis=0, keepdims=True)
    ce = jnp.sum(ce_ref[:, 0, :], axis=0, keepdims=True)
    hot_value = n_expert / n_tokens
    loss_ref[...] = jnp.sum(me * ce, axis=1, keepdims=True) * (
        hot_value / n_tokens)


def kernel(inp, W):
    n_tokens, d_model = inp.shape
    n_expert = W.shape[0]
    tile = 1024
    while n_tokens % tile:
        tile //= 2
    grid = n_tokens // tile

    idx, score, me_parts, ce_parts = pl.pallas_call(
        _router_body,
        grid=(grid,),
        in_specs=[
            pl.BlockSpec((tile, d_model), lambda i: (i, 0)),
            pl.BlockSpec((n_expert, d_model), lambda i: (0, 0)),
        ],
        out_specs=[
            pl.BlockSpec((tile, _TOP_K), lambda i: (i, 0)),
            pl.BlockSpec((tile, _TOP_K), lambda i: (i, 0)),
            pl.BlockSpec((1, 1, n_expert), lambda i: (i, 0, 0)),
            pl.BlockSpec((1, 1, n_expert), lambda i: (i, 0, 0)),
        ],
        out_shape=[
            jax.ShapeDtypeStruct((n_tokens, _TOP_K), jnp.int32),
            jax.ShapeDtypeStruct((n_tokens, _TOP_K), jnp.float32),
            jax.ShapeDtypeStruct((grid, 1, n_expert), jnp.float32),
            jax.ShapeDtypeStruct((grid, 1, n_expert), jnp.float32),
        ],
        compiler_params=pltpu.CompilerParams(
            dimension_semantics=("parallel",),
        ),
    )(inp, W)

    loss = pl.pallas_call(
        functools.partial(_loss_body, n_tokens=n_tokens, n_expert=n_expert),
        out_shape=jax.ShapeDtypeStruct((1, 1), jnp.float32),
    )(me_parts, ce_parts)
    return idx, score, loss.reshape(())
